# baseline (speedup 1.0000x reference)
import functools

import jax
import jax.numpy as jnp
from jax import lax
from jax.experimental import pallas as pl
from jax.experimental.pallas import tpu as pltpu
from jax.experimental.pallas import tpu_sc as plsc

POOL = 16
B, C, L = 4, 768, 4096
NUM_WORKERS = 32
W_PER_B = NUM_WORKERS // B
ROWS_PER_W = C // W_PER_B
CHUNK_ROWS = 8
CHUNK_COLS = 2048
COL_SPLITS = L // CHUNK_COLS
NCHUNKS = (ROWS_PER_W // CHUNK_ROWS) * COL_SPLITS


def _extrema_body(x_hbm, out_hbm, in0, in1, out0, out1, si0, si1, so0, so1):
    cid = lax.axis_index("c")
    sid = lax.axis_index("s")
    wid = sid * 2 + cid
    b_idx = wid // W_PER_B
    row_base = (wid % W_PER_B) * ROWS_PER_W
    lanes = lax.iota(jnp.int32, POOL)
    ins, outs = (in0, in1), (out0, out1)
    sis, sos = (si0, si1), (so0, so1)

    def slices(ci):
        row = row_base + (ci // COL_SPLITS) * CHUNK_ROWS
        col = (ci % COL_SPLITS) * CHUNK_COLS
        return pl.ds(row, CHUNK_ROWS), pl.ds(col, CHUNK_COLS)

    def in_copy(ci, b):
        rs, cs = slices(ci)
        return pltpu.make_async_copy(x_hbm.at[b_idx, rs, cs], ins[b], sis[b])

    def out_copy(ci, b):
        rs, cs = slices(ci)
        return pltpu.make_async_copy(outs[b], out_hbm.at[b_idx, rs, cs], sos[b])

    in_copy(0, 0).start()
    in_copy(1, 1).start()

    def pair_body(p, carry):
        for b in range(2):
            ci = 2 * p + b
            in_copy(ci, b).wait()

            @pl.when(ci >= 2)
            def _():
                out_copy(ci - 2, b).wait()

            for r in range(CHUNK_ROWS):
                @plsc.parallel_loop(0, CHUNK_COLS, step=POOL, unroll=16)
                def win_body(coff):
                    w = ins[b][r, pl.ds(coff, POOL)]
                    a = jnp.abs(w)
                    mx = jnp.max(a)
                    first = plsc.all_reduce_ffs(a == mx)
                    outs[b][r, pl.ds(coff, POOL)] = jnp.where(
                        lanes == first, w, 0.0)

            out_copy(ci, b).start()

            @pl.when(ci + 2 < NCHUNKS)
            def _():
                in_copy(ci + 2, b).start()

        return carry

    lax.fori_loop(0, NCHUNKS // 2, pair_body, 0)
    out_copy(NCHUNKS - 2, 0).wait()
    out_copy(NCHUNKS - 1, 1).wait()


def kernel(input_):
    mesh = plsc.VectorSubcoreMesh(core_axis_name="c", subcore_axis_name="s")
    return pl.kernel(
        _extrema_body,
        mesh=mesh,
        out_type=jax.ShapeDtypeStruct((B, C, L), jnp.float32),
        scratch_types=[
            pltpu.VMEM((CHUNK_ROWS, CHUNK_COLS), jnp.float32),
            pltpu.VMEM((CHUNK_ROWS, CHUNK_COLS), jnp.float32),
            pltpu.VMEM((CHUNK_ROWS, CHUNK_COLS), jnp.float32),
            pltpu.VMEM((CHUNK_ROWS, CHUNK_COLS), jnp.float32),
            pltpu.SemaphoreType.DMA,
            pltpu.SemaphoreType.DMA,
            pltpu.SemaphoreType.DMA,
            pltpu.SemaphoreType.DMA,
        ],
        compiler_params=pltpu.CompilerParams(needs_layout_passes=False),
    )(input_)


# speedup vs baseline: 1.2180x; 1.2180x over previous
import functools

import jax
import jax.numpy as jnp
from jax import lax
from jax.experimental import pallas as pl
from jax.experimental.pallas import tpu as pltpu
from jax.experimental.pallas import tpu_sc as plsc

POOL = 16
B, C, L = 4, 768, 4096
NUM_WORKERS = 32
W_PER_B = NUM_WORKERS // B
ROWS_PER_W = C // W_PER_B
CHUNK_ROWS = 4
NCHUNKS = ROWS_PER_W // CHUNK_ROWS
DEPTH = 3


def _extrema_body(x_hbm, out_hbm,
                  in0, in1, in2, out0, out1, out2,
                  si0, si1, si2, so0, so1, so2):
    cid = lax.axis_index("c")
    sid = lax.axis_index("s")
    wid = sid * 2 + cid
    b_idx = wid // W_PER_B
    row_base = (wid % W_PER_B) * ROWS_PER_W
    lanes = lax.iota(jnp.int32, POOL)
    ins, outs = (in0, in1, in2), (out0, out1, out2)
    sis, sos = (si0, si1, si2), (so0, so1, so2)

    def in_copy(ci, b):
        return pltpu.make_async_copy(
            x_hbm.at[b_idx, pl.ds(row_base + ci * CHUNK_ROWS, CHUNK_ROWS), :],
            ins[b], sis[b])

    def out_copy(ci, b):
        return pltpu.make_async_copy(
            outs[b],
            out_hbm.at[b_idx, pl.ds(row_base + ci * CHUNK_ROWS, CHUNK_ROWS), :],
            sos[b])

    for b in range(DEPTH):
        in_copy(b, b).start()

    def ring_body(p, carry):
        for b in range(DEPTH):
            ci = DEPTH * p + b
            in_copy(ci, b).wait()

            @pl.when(ci >= DEPTH)
            def _():
                out_copy(ci - DEPTH, b).wait()

            for r in range(CHUNK_ROWS):
                @plsc.parallel_loop(0, L, step=POOL, unroll=16)
                def win_body(coff):
                    w = ins[b][r, pl.ds(coff, POOL)]
                    a = jnp.abs(w)
                    mx = jnp.max(a)
                    first = plsc.all_reduce_ffs(a == mx)
                    outs[b][r, pl.ds(coff, POOL)] = jnp.where(
                        lanes == first, w, 0.0)

            out_copy(ci, b).start()

            @pl.when(ci + DEPTH < NCHUNKS)
            def _():
                in_copy(ci + DEPTH, b).start()

        return carry

    lax.fori_loop(0, NCHUNKS // DEPTH, ring_body, 0)
    for b in range(DEPTH):
        out_copy(NCHUNKS - DEPTH + b, b).wait()


def kernel(input_):
    mesh = plsc.VectorSubcoreMesh(core_axis_name="c", subcore_axis_name="s")
    return pl.kernel(
        _extrema_body,
        mesh=mesh,
        out_type=jax.ShapeDtypeStruct((B, C, L), jnp.float32),
        scratch_types=[
            pltpu.VMEM((CHUNK_ROWS, L), jnp.float32),
            pltpu.VMEM((CHUNK_ROWS, L), jnp.float32),
            pltpu.VMEM((CHUNK_ROWS, L), jnp.float32),
            pltpu.VMEM((CHUNK_ROWS, L), jnp.float32),
            pltpu.VMEM((CHUNK_ROWS, L), jnp.float32),
            pltpu.VMEM((CHUNK_ROWS, L), jnp.float32),
            pltpu.SemaphoreType.DMA,
            pltpu.SemaphoreType.DMA,
            pltpu.SemaphoreType.DMA,
            pltpu.SemaphoreType.DMA,
            pltpu.SemaphoreType.DMA,
            pltpu.SemaphoreType.DMA,
        ],
        compiler_params=pltpu.CompilerParams(needs_layout_passes=False),
    )(input_)
